# Initial kernel scaffold; baseline (speedup 1.0000x reference)
import jax
import jax.numpy as jnp
from jax.experimental import pallas as pl


def kernel(arg0_1):
    raise NotImplementedError("write your pallas kernel here")


# speedup vs baseline: 1.3128x; 1.3128x over previous
import functools

import jax
import jax.numpy as jnp
import numpy as np
from jax import lax
from jax.experimental import pallas as pl
from jax.experimental.pallas import tpu as pltpu
from jax.experimental.pallas import tpu_sc as plsc

OUT_W = 272
IN_W = 456
SCALE = 1.6764705882352942
INV_SUPPORT = 0.5964912280701754
NTAPS = 4

B, C, H = 64, 3, 456
R = B * C * H
NWORKERS = 32
ROWS_PER_W = R // NWORKERS
CHUNK = 48
NCHUNKS = ROWS_PER_W // CHUNK
NOVEC = OUT_W // 16


def _make_tables():
    i = np.arange(OUT_W, dtype=np.float32)
    center = (i + np.float32(0.5)) * np.float32(SCALE)
    low = np.clip((center - np.float32(SCALE) + np.float32(0.5)).astype(np.int32), 0, None)
    high = np.minimum((center + np.float32(SCALE) + np.float32(0.5)).astype(np.int32), IN_W)
    width = np.minimum(high - low, 5)
    j = np.arange(5)
    dist = (j[None, :].astype(np.float32) + low[:, None].astype(np.float32)
            - center[:, None] + np.float32(0.5)) * np.float32(INV_SUPPORT)
    w = np.float32(1.0) - np.minimum(np.abs(dist), np.float32(1.0))
    w = np.where(j[None, :] < width[:, None], w, np.float32(0.0))
    w = w / w.sum(axis=-1, keepdims=True)
    idx = np.minimum(low[:, None] + j[None, :], IN_W - 1)
    return (np.ascontiguousarray(idx[:, :NTAPS].T.astype(np.int32)),
            np.ascontiguousarray(w[:, :NTAPS].T.astype(np.float32)))


_IDX_T, _W_T = _make_tables()


def _resize_body(x_hbm, idx_hbm, w_hbm, out_hbm, idx_v, w_v, in_v, out_v):
    wid = lax.axis_index("s") * 2 + lax.axis_index("c")
    base_row = wid * ROWS_PER_W
    pltpu.sync_copy(idx_hbm, idx_v)
    pltpu.sync_copy(w_hbm, w_v)

    def chunk_body(ci, carry):
        row0 = base_row + ci * CHUNK
        pltpu.sync_copy(x_hbm.at[pl.ds(row0, CHUNK), :], in_v)
        for og in range(NOVEC):
            idxs = [idx_v[j, pl.ds(og * 16, 16)] for j in range(NTAPS)]
            ws = [w_v[j, pl.ds(og * 16, 16)] for j in range(NTAPS)]

            def row_body(r, c2, idxs=idxs, ws=ws, og=og):
                rsplat = jnp.full((16,), r, dtype=jnp.int32)
                acc = plsc.load_gather(in_v, [rsplat, idxs[0]]) * ws[0]
                for j in range(1, NTAPS):
                    acc = acc + plsc.load_gather(in_v, [rsplat, idxs[j]]) * ws[j]
                out_v[r, pl.ds(og * 16, 16)] = acc
                return c2

            lax.fori_loop(0, CHUNK, row_body, 0, unroll=2)
        pltpu.sync_copy(out_v, out_hbm.at[pl.ds(row0, CHUNK), :])
        return carry

    lax.fori_loop(0, NCHUNKS, chunk_body, 0)


@jax.jit
def _resize(x2d, idx_t, w_t):
    mesh = plsc.VectorSubcoreMesh(core_axis_name="c", subcore_axis_name="s")
    return pl.kernel(
        _resize_body,
        out_type=jax.ShapeDtypeStruct((R, OUT_W), jnp.float32),
        mesh=mesh,
        compiler_params=pltpu.CompilerParams(
            use_tc_tiling_on_sc=False, needs_layout_passes=False),
        scratch_types=[
            pltpu.VMEM((NTAPS, OUT_W), jnp.int32),
            pltpu.VMEM((NTAPS, OUT_W), jnp.float32),
            pltpu.VMEM((CHUNK, IN_W), jnp.float32),
            pltpu.VMEM((CHUNK, OUT_W), jnp.float32),
        ],
    )(x2d, idx_t, w_t)


def kernel(arg0_1):
    x2d = arg0_1.reshape(R, IN_W)
    out2d = _resize(x2d, jnp.asarray(_IDX_T), jnp.asarray(_W_T))
    return (out2d.reshape(B, C, H, OUT_W),)


# speedup vs baseline: 1.3197x; 1.0053x over previous
import jax
import jax.numpy as jnp
import numpy as np
from jax import lax
from jax.experimental import pallas as pl
from jax.experimental.pallas import tpu as pltpu
from jax.experimental.pallas import tpu_sc as plsc

OUT_W = 272
IN_W = 456
SCALE = 1.6764705882352942
INV_SUPPORT = 0.5964912280701754
NTAPS = 4

B, C, H = 64, 3, 456
R = B * C * H
NWORKERS = 32
ROWS_PER_W = R // NWORKERS
CHUNK = 48
NCHUNKS = ROWS_PER_W // CHUNK
NOVEC = OUT_W // 16


def _make_tables():
    i = np.arange(OUT_W, dtype=np.float32)
    center = (i + np.float32(0.5)) * np.float32(SCALE)
    low = np.clip((center - np.float32(SCALE) + np.float32(0.5)).astype(np.int32), 0, None)
    high = np.minimum((center + np.float32(SCALE) + np.float32(0.5)).astype(np.int32), IN_W)
    width = np.minimum(high - low, 5)
    j = np.arange(5)
    dist = (j[None, :].astype(np.float32) + low[:, None].astype(np.float32)
            - center[:, None] + np.float32(0.5)) * np.float32(INV_SUPPORT)
    w = np.float32(1.0) - np.minimum(np.abs(dist), np.float32(1.0))
    w = np.where(j[None, :] < width[:, None], w, np.float32(0.0))
    w = w / w.sum(axis=-1, keepdims=True)
    idx = np.minimum(low[:, None] + j[None, :], IN_W - 1)
    return (np.ascontiguousarray(idx[:, :NTAPS].T.astype(np.int32)).reshape(-1),
            np.ascontiguousarray(w[:, :NTAPS].T.astype(np.float32)).reshape(-1))


_IDX_T, _W_T = _make_tables()


def _resize_body(x_hbm, idx_hbm, w_hbm, out_hbm, idx_v, w_v, in_v, out_v):
    wid = lax.axis_index("s") * 2 + lax.axis_index("c")
    base_row = wid * ROWS_PER_W
    pltpu.sync_copy(idx_hbm, idx_v)
    pltpu.sync_copy(w_hbm, w_v)
    row_step = jnp.full((16,), IN_W, dtype=jnp.int32)

    def chunk_body(ci, carry):
        row0 = base_row + ci * CHUNK
        pltpu.sync_copy(x_hbm.at[pl.ds(row0 * IN_W, CHUNK * IN_W)], in_v)
        for og in range(NOVEC):
            idx0 = [idx_v[pl.ds(j * OUT_W + og * 16, 16)] for j in range(NTAPS)]
            ws = [w_v[pl.ds(j * OUT_W + og * 16, 16)] for j in range(NTAPS)]

            def row_body(r, idxs, ws=ws, og=og):
                acc = plsc.load_gather(in_v, [idxs[0]]) * ws[0]
                for j in range(1, NTAPS):
                    acc = acc + plsc.load_gather(in_v, [idxs[j]]) * ws[j]
                out_v[pl.ds(r * OUT_W + og * 16, 16)] = acc
                return tuple(ix + row_step for ix in idxs)

            lax.fori_loop(0, CHUNK, row_body, tuple(idx0), unroll=4)
        pltpu.sync_copy(out_v, out_hbm.at[pl.ds(row0 * OUT_W, CHUNK * OUT_W)])
        return carry

    lax.fori_loop(0, NCHUNKS, chunk_body, 0)


@jax.jit
def _resize(x1d, idx_t, w_t):
    mesh = plsc.VectorSubcoreMesh(core_axis_name="c", subcore_axis_name="s")
    return pl.kernel(
        _resize_body,
        out_type=jax.ShapeDtypeStruct((R * OUT_W,), jnp.float32),
        mesh=mesh,
        compiler_params=pltpu.CompilerParams(
            use_tc_tiling_on_sc=False, needs_layout_passes=False),
        scratch_types=[
            pltpu.VMEM((NTAPS * OUT_W,), jnp.int32),
            pltpu.VMEM((NTAPS * OUT_W,), jnp.float32),
            pltpu.VMEM((CHUNK * IN_W,), jnp.float32),
            pltpu.VMEM((CHUNK * OUT_W,), jnp.float32),
        ],
    )(x1d, idx_t, w_t)


def kernel(arg0_1):
    x1d = arg0_1.reshape(R * IN_W)
    out1d = _resize(x1d, jnp.asarray(_IDX_T), jnp.asarray(_W_T))
    return (out1d.reshape(B, C, H, OUT_W),)


# speedup vs baseline: 1.3264x; 1.0051x over previous
import jax
import jax.numpy as jnp
import numpy as np
from jax import lax
from jax.experimental import pallas as pl
from jax.experimental.pallas import tpu as pltpu
from jax.experimental.pallas import tpu_sc as plsc

OUT_W = 272
IN_W = 456
SCALE = 1.6764705882352942
INV_SUPPORT = 0.5964912280701754
NTAPS = 4

B, C, H = 64, 3, 456
R = B * C * H
NWORKERS = 32
ROWS_PER_W = R // NWORKERS
CHUNK = 48
NCHUNKS = ROWS_PER_W // CHUNK
NOVEC = OUT_W // 16


def _make_tables():
    i = np.arange(OUT_W, dtype=np.float32)
    center = (i + np.float32(0.5)) * np.float32(SCALE)
    low = np.clip((center - np.float32(SCALE) + np.float32(0.5)).astype(np.int32), 0, None)
    high = np.minimum((center + np.float32(SCALE) + np.float32(0.5)).astype(np.int32), IN_W)
    width = np.minimum(high - low, 5)
    j = np.arange(5)
    dist = (j[None, :].astype(np.float32) + low[:, None].astype(np.float32)
            - center[:, None] + np.float32(0.5)) * np.float32(INV_SUPPORT)
    w = np.float32(1.0) - np.minimum(np.abs(dist), np.float32(1.0))
    w = np.where(j[None, :] < width[:, None], w, np.float32(0.0))
    w = w / w.sum(axis=-1, keepdims=True)
    idx = np.minimum(low[:, None] + j[None, :], IN_W - 1)
    return (np.ascontiguousarray(idx[:, :NTAPS].T.astype(np.int32)).reshape(-1),
            np.ascontiguousarray(w[:, :NTAPS].T.astype(np.float32)).reshape(-1))


_IDX_T, _W_T = _make_tables()


def _resize_body(x_hbm, idx_hbm, w_hbm, out_hbm, idx_v, w_v, in_v, out_v):
    wid = lax.axis_index("s") * 2 + lax.axis_index("c")
    base_row = wid * ROWS_PER_W
    pltpu.sync_copy(idx_hbm, idx_v)
    pltpu.sync_copy(w_hbm, w_v)
    row_step = jnp.full((16,), IN_W, dtype=jnp.int32)

    def chunk_body(ci, carry):
        row0 = base_row + ci * CHUNK
        pltpu.sync_copy(x_hbm.at[pl.ds(row0 * IN_W, CHUNK * IN_W)], in_v)
        for og in range(NOVEC):
            idx0 = [idx_v[pl.ds(j * OUT_W + og * 16, 16)] for j in range(NTAPS)]
            ws = [w_v[pl.ds(j * OUT_W + og * 16, 16)] for j in range(NTAPS)]

            def row_body(r, idxs, ws=ws, og=og):
                g = [plsc.load_gather(in_v, [ix]) for ix in idxs]
                acc = (g[0] * ws[0] + g[1] * ws[1]) + (g[2] * ws[2] + g[3] * ws[3])
                out_v[pl.ds(r * OUT_W + og * 16, 16)] = acc
                return tuple(ix + row_step for ix in idxs)

            lax.fori_loop(0, CHUNK, row_body, tuple(idx0), unroll=4)
        pltpu.sync_copy(out_v, out_hbm.at[pl.ds(row0 * OUT_W, CHUNK * OUT_W)])
        return carry

    lax.fori_loop(0, NCHUNKS, chunk_body, 0)


@jax.jit
def _resize(x1d, idx_t, w_t):
    mesh = plsc.VectorSubcoreMesh(core_axis_name="c", subcore_axis_name="s")
    return pl.kernel(
        _resize_body,
        out_type=jax.ShapeDtypeStruct((R * OUT_W,), jnp.float32),
        mesh=mesh,
        compiler_params=pltpu.CompilerParams(needs_layout_passes=False),
        scratch_types=[
            pltpu.VMEM((NTAPS * OUT_W,), jnp.int32),
            pltpu.VMEM((NTAPS * OUT_W,), jnp.float32),
            pltpu.VMEM((CHUNK * IN_W,), jnp.float32),
            pltpu.VMEM((CHUNK * OUT_W,), jnp.float32),
        ],
    )(x1d, idx_t, w_t)


def kernel(arg0_1):
    x1d = arg0_1.reshape(R * IN_W)
    out1d = _resize(x1d, jnp.asarray(_IDX_T), jnp.asarray(_W_T))
    return (out1d.reshape(B, C, H, OUT_W),)


# speedup vs baseline: 1.7346x; 1.3077x over previous
import jax
import jax.numpy as jnp
from jax import lax
from jax.experimental import pallas as pl
from jax.experimental.pallas import tpu as pltpu
from jax.experimental.pallas import tpu_sc as plsc

OUT_W = 272
IN_W = 456
SCALE = 1.6764705882352942
INV_SUPPORT = 0.5964912280701754
NTAPS = 4

B, C, H = 64, 3, 456
R = B * C * H
NWORKERS = 32
ROWS_PER_W = R // NWORKERS
CHUNK = 48
NCHUNKS = ROWS_PER_W // CHUNK
NOVEC = OUT_W // 16


def _tap_tables(og):
    f32, i32 = jnp.float32, jnp.int32
    o = (lax.iota(i32, 16) + og * 16).astype(f32)
    center = (o + 0.5) * SCALE
    lowi = jnp.maximum((center - SCALE + 0.5).astype(i32), 0)
    highi = jnp.minimum((center + SCALE + 0.5).astype(i32), IN_W)
    width = jnp.minimum(highi - lowi, 5)
    lowf = lowi.astype(f32)
    ws, idxs = [], []
    for j in range(NTAPS):
        dist = (lowf + float(j) - center + 0.5) * INV_SUPPORT
        wj = 1.0 - jnp.minimum(jnp.abs(dist), 1.0)
        wj = jnp.where(width > j, wj, 0.0)
        ws.append(wj)
        idxs.append(jnp.minimum(lowi + j, IN_W - 1))
    wsum = (ws[0] + ws[1]) + (ws[2] + ws[3])
    ws = [w / wsum for w in ws]
    return idxs, ws


def _resize_body(x_hbm, out_hbm, in_v, out_v):
    wid = lax.axis_index("s") * 2 + lax.axis_index("c")
    base_row = wid * ROWS_PER_W

    def chunk_body(ci, carry):
        row0 = base_row + ci * CHUNK
        pltpu.sync_copy(x_hbm.at[pl.ds(row0, CHUNK), :], in_v)
        for og in range(NOVEC):
            idxs, ws = _tap_tables(og)

            def row_body(r, c2, idxs=idxs, ws=ws, og=og):
                rv = jnp.full((16,), r, dtype=jnp.int32)
                g = [plsc.load_gather(in_v, [rv, ix]) for ix in idxs]
                acc = (g[0] * ws[0] + g[1] * ws[1]) + (g[2] * ws[2] + g[3] * ws[3])
                out_v[r, pl.ds(og * 16, 16)] = acc
                return c2

            lax.fori_loop(0, CHUNK, row_body, 0, unroll=4)
        pltpu.sync_copy(out_v, out_hbm.at[pl.ds(row0, CHUNK), :])
        return carry

    lax.fori_loop(0, NCHUNKS, chunk_body, 0)


@jax.jit
def _resize(x2d):
    mesh = plsc.VectorSubcoreMesh(core_axis_name="c", subcore_axis_name="s")
    return pl.kernel(
        _resize_body,
        out_type=jax.ShapeDtypeStruct((R, OUT_W), jnp.float32),
        mesh=mesh,
        compiler_params=pltpu.CompilerParams(needs_layout_passes=False),
        scratch_types=[
            pltpu.VMEM((CHUNK, IN_W), jnp.float32),
            pltpu.VMEM((CHUNK, OUT_W), jnp.float32),
        ],
    )(x2d)


def kernel(arg0_1):
    x2d = arg0_1.reshape(R, IN_W)
    out2d = _resize(x2d)
    return (out2d.reshape(B, C, H, OUT_W),)


# speedup vs baseline: 1.7629x; 1.0164x over previous
import jax
import jax.numpy as jnp
from jax import lax
from jax.experimental import pallas as pl
from jax.experimental.pallas import tpu as pltpu
from jax.experimental.pallas import tpu_sc as plsc

OUT_W = 272
IN_W = 456
SCALE = 1.6764705882352942
INV_SUPPORT = 0.5964912280701754
NTAPS = 4

B, C, H = 64, 3, 456
R = B * C * H
NWORKERS = 32
ROWS_PER_W = R // NWORKERS
CHUNK = 24
NCHUNKS = ROWS_PER_W // CHUNK
NOVEC = OUT_W // 16


def _tap_tables(og):
    f32, i32 = jnp.float32, jnp.int32
    o = (lax.iota(i32, 16) + og * 16).astype(f32)
    center = (o + 0.5) * SCALE
    lowi = jnp.maximum((center - SCALE + 0.5).astype(i32), 0)
    highi = jnp.minimum((center + SCALE + 0.5).astype(i32), IN_W)
    width = jnp.minimum(highi - lowi, 5)
    lowf = lowi.astype(f32)
    ws, idxs = [], []
    for j in range(NTAPS):
        dist = (lowf + float(j) - center + 0.5) * INV_SUPPORT
        wj = 1.0 - jnp.minimum(jnp.abs(dist), 1.0)
        wj = jnp.where(width > j, wj, 0.0)
        ws.append(wj)
        idxs.append(jnp.minimum(lowi + j, IN_W - 1))
    wsum = (ws[0] + ws[1]) + (ws[2] + ws[3])
    ws = [w / wsum for w in ws]
    return idxs, ws


def _resize_body(x_hbm, out_hbm, idx_t, w_t, in_v0, in_v1, out_v0, out_v1,
                 sin0, sin1, sout0, sout1):
    wid = lax.axis_index("s") * 2 + lax.axis_index("c")
    base_row = wid * ROWS_PER_W
    in_bufs = (in_v0, in_v1)
    out_bufs = (out_v0, out_v1)
    sins = (sin0, sin1)
    souts = (sout0, sout1)

    for og in range(NOVEC):
        idxs, ws = _tap_tables(og)
        for j in range(NTAPS):
            idx_t[j, pl.ds(og * 16, 16)] = idxs[j]
            w_t[j, pl.ds(og * 16, 16)] = ws[j]

    def in_copy(ci, b):
        row0 = base_row + ci * CHUNK
        return pltpu.make_async_copy(
            x_hbm.at[pl.ds(row0, CHUNK), :], in_bufs[b], sins[b])

    def out_copy(ci, b):
        row0 = base_row + ci * CHUNK
        return pltpu.make_async_copy(
            out_bufs[b], out_hbm.at[pl.ds(row0, CHUNK), :], souts[b])

    in_copy(0, 0).start()

    def outer(cc, carry):
        for b in range(2):
            ci = cc * 2 + b

            @pl.when(ci + 1 < NCHUNKS)
            def _():
                in_copy(ci + 1, 1 - b).start()

            in_copy(ci, b).wait()

            @pl.when(cc >= 1)
            def _():
                out_copy(ci - 2, b).wait()

            for og in range(NOVEC):
                colv = [idx_t[j, pl.ds(og * 16, 16)] for j in range(NTAPS)]
                wv = [w_t[j, pl.ds(og * 16, 16)] for j in range(NTAPS)]

                def row_body(r, c2, colv=colv, wv=wv, og=og, b=b):
                    rv = jnp.full((16,), r, dtype=jnp.int32)
                    g = [plsc.load_gather(in_bufs[b], [rv, cv]) for cv in colv]
                    acc = (g[0] * wv[0] + g[1] * wv[1]) + (g[2] * wv[2] + g[3] * wv[3])
                    out_bufs[b][r, pl.ds(og * 16, 16)] = acc
                    return c2

                lax.fori_loop(0, CHUNK, row_body, 0, unroll=4)

            out_copy(ci, b).start()
        return carry

    lax.fori_loop(0, NCHUNKS // 2, outer, 0)
    out_copy(NCHUNKS - 2, 0).wait()
    out_copy(NCHUNKS - 1, 1).wait()


@jax.jit
def _resize(x2d):
    mesh = plsc.VectorSubcoreMesh(core_axis_name="c", subcore_axis_name="s")
    return pl.kernel(
        _resize_body,
        out_type=jax.ShapeDtypeStruct((R, OUT_W), jnp.float32),
        mesh=mesh,
        compiler_params=pltpu.CompilerParams(needs_layout_passes=False),
        scratch_types=[
            pltpu.VMEM((NTAPS, OUT_W), jnp.int32),
            pltpu.VMEM((NTAPS, OUT_W), jnp.float32),
            pltpu.VMEM((CHUNK, IN_W), jnp.float32),
            pltpu.VMEM((CHUNK, IN_W), jnp.float32),
            pltpu.VMEM((CHUNK, OUT_W), jnp.float32),
            pltpu.VMEM((CHUNK, OUT_W), jnp.float32),
            pltpu.SemaphoreType.DMA,
            pltpu.SemaphoreType.DMA,
            pltpu.SemaphoreType.DMA,
            pltpu.SemaphoreType.DMA,
        ],
    )(x2d)


def kernel(arg0_1):
    x2d = arg0_1.reshape(R, IN_W)
    out2d = _resize(x2d)
    return (out2d.reshape(B, C, H, OUT_W),)
